# Initial kernel scaffold; baseline (speedup 1.0000x reference)
import jax
import jax.numpy as jnp
from jax.experimental import pallas as pl


def kernel(x, table):
    raise NotImplementedError("write your pallas kernel here")


# speedup vs baseline: 1.3424x; 1.3424x over previous
import functools

import jax
import jax.numpy as jnp
from jax import lax
from jax.experimental import pallas as pl
from jax.experimental.pallas import tpu as pltpu
from jax.experimental.pallas import tpu_sc as plsc

D_MODEL = 128
BATCH = 1024
SEQ = 200

_NC = 2
_NS = 16
_NW = _NC * _NS
_N_ROWS = BATCH * SEQ
_ROWS_PER_W = _N_ROWS // _NW
_CHUNK = 100
_NCHUNK = _ROWS_PER_W // _CHUNK
_NBUF = 2
_L = 16


def _pos_encoding():
    pos = jnp.arange(SEQ, dtype=jnp.float32)[:, None]
    i_even = jnp.arange(0, D_MODEL, 2, dtype=jnp.float32)[None, :]
    angles = pos / jnp.power(10000.0, i_even / D_MODEL)
    enc = jnp.zeros((SEQ, D_MODEL), dtype=jnp.float32)
    enc = enc.at[:, 0::2].set(jnp.sin(angles))
    enc = enc.at[:, 1::2].set(jnp.cos(angles))
    return enc


def _body(idx_hbm, table_hbm, pos_hbm, out_hbm, idx_v, pos_v, rows_v, *sems):
    wid = lax.axis_index("s") * _NC + lax.axis_index("c")
    row0 = wid * _ROWS_PER_W

    pltpu.sync_copy(idx_hbm.at[pl.ds(wid * _NCHUNK, _NCHUNK)], idx_v)
    pltpu.sync_copy(pos_hbm, pos_v)

    def start_gather(c, b):
        pltpu.make_async_copy(
            table_hbm.at[idx_v.at[c]], rows_v.at[b], sems[b]
        ).start()

    def wait_gather(b):
        pltpu.make_async_copy(
            table_hbm.at[idx_v.at[0]], rows_v.at[b], sems[b]
        ).wait()

    for b in range(_NBUF):
        start_gather(b, b)

    def round_body(g, carry):
        for b in range(_NBUF):
            c = g * _NBUF + b
            wait_gather(b)

            pbase = lax.rem(c, 2) * _CHUNK

            def add_row(r, carry2):
                for j in range(D_MODEL // _L):
                    v = pos_v[pbase + r, pl.ds(j * _L, _L)]
                    plsc.addupdate(rows_v.at[b, r, pl.ds(j * _L, _L)], v)
                return carry2

            lax.fori_loop(0, _CHUNK, add_row, 0)

            pltpu.sync_copy(
                rows_v.at[b], out_hbm.at[pl.ds(row0 + c * _CHUNK, _CHUNK)]
            )

            nxt = c + _NBUF

            @pl.when(nxt < _NCHUNK)
            def _():
                start_gather(nxt, b)

        return carry

    lax.fori_loop(0, _NCHUNK // _NBUF, round_body, 0)


@jax.jit
def _emb(x, table):
    pos = _pos_encoding()
    xi = x.reshape(_N_ROWS // _CHUNK, _CHUNK).astype(jnp.int32)
    run = pl.kernel(
        _body,
        mesh=plsc.VectorSubcoreMesh(core_axis_name="c", subcore_axis_name="s"),
        compiler_params=pltpu.CompilerParams(use_tc_tiling_on_sc=False),
        out_type=jax.ShapeDtypeStruct((_N_ROWS, D_MODEL), jnp.float32),
        scratch_types=[
            pltpu.VMEM((_NCHUNK, _CHUNK), jnp.int32),
            pltpu.VMEM((SEQ, D_MODEL), jnp.float32),
            pltpu.VMEM((_NBUF, _CHUNK, D_MODEL), jnp.float32),
        ] + [pltpu.SemaphoreType.DMA] * _NBUF,
    )
    out = run(xi, table, pos)
    return out.reshape(BATCH, SEQ, D_MODEL)


def kernel(x, table):
    return _emb(x, table)


# speedup vs baseline: 1.4872x; 1.1078x over previous
import functools

import jax
import jax.numpy as jnp
from jax import lax
from jax.experimental import pallas as pl
from jax.experimental.pallas import tpu as pltpu
from jax.experimental.pallas import tpu_sc as plsc

D_MODEL = 128
BATCH = 1024
SEQ = 200

_NC = 2
_NS = 16
_NW = _NC * _NS
_N_ROWS = BATCH * SEQ
_ROWS_PER_W = _N_ROWS // _NW
_CHUNK = 100
_NCHUNK = _ROWS_PER_W // _CHUNK
_NBUF = 4
_L = 16


def _pos_encoding():
    pos = jnp.arange(SEQ, dtype=jnp.float32)[:, None]
    i_even = jnp.arange(0, D_MODEL, 2, dtype=jnp.float32)[None, :]
    angles = pos / jnp.power(10000.0, i_even / D_MODEL)
    enc = jnp.zeros((SEQ, D_MODEL), dtype=jnp.float32)
    enc = enc.at[:, 0::2].set(jnp.sin(angles))
    enc = enc.at[:, 1::2].set(jnp.cos(angles))
    return enc


def _body(idx_hbm, table_hbm, pos_hbm, out_hbm, idx_v, pos_v, rows_v, *sems):
    gsems = sems[:_NBUF]
    ssems = sems[_NBUF:]
    wid = lax.axis_index("s") * _NC + lax.axis_index("c")
    row0 = wid * _ROWS_PER_W

    pltpu.sync_copy(idx_hbm.at[pl.ds(wid * _NCHUNK, _NCHUNK)], idx_v)

    def start_gather(c, b):
        pltpu.make_async_copy(
            table_hbm.at[idx_v.at[c]], rows_v.at[b], gsems[b]
        ).start()

    def wait_gather(b):
        pltpu.make_async_copy(
            table_hbm.at[idx_v.at[0]], rows_v.at[b], gsems[b]
        ).wait()

    def start_store(c, b):
        pltpu.make_async_copy(
            rows_v.at[b], out_hbm.at[pl.ds(row0 + c * _CHUNK, _CHUNK)], ssems[b]
        ).start()

    def wait_store(b):
        pltpu.make_async_copy(
            rows_v.at[b], out_hbm.at[pl.ds(row0, _CHUNK)], ssems[b]
        ).wait()

    def add_pos(c, b):
        pbase = lax.rem(c, 2) * _CHUNK

        def add_row(r, carry2):
            for j in range(D_MODEL // _L):
                v = pos_v[pbase + r, pl.ds(j * _L, _L)]
                plsc.addupdate(rows_v.at[b, r, pl.ds(j * _L, _L)], v)
            return carry2

        lax.fori_loop(0, _CHUNK, add_row, 0, unroll=2)

    start_gather(0, 0)
    start_gather(1, 1)
    pltpu.sync_copy(pos_hbm, pos_v)

    for b in range(_NBUF):
        wait_gather(b)
        add_pos(b, b)
        start_store(b, b)
        nxt = b + 2
        if nxt < _NBUF:
            start_gather(nxt, nxt)
        else:
            b2 = nxt % _NBUF
            wait_store(b2)
            start_gather(nxt, b2)

    def round_body(g, carry):
        for b in range(_NBUF):
            c = g * _NBUF + b
            wait_gather(b)
            add_pos(c, b)
            start_store(c, b)
            nxt = c + 2
            b2 = (b + 2) % _NBUF

            @pl.when(nxt < _NCHUNK)
            def _():
                wait_store(b2)
                start_gather(nxt, b2)

        return carry

    lax.fori_loop(1, _NCHUNK // _NBUF, round_body, 0)

    for b in range(_NBUF):
        wait_store(b)


@jax.jit
def _emb(x, table):
    pos = _pos_encoding()
    xi = x.reshape(_N_ROWS // _CHUNK, _CHUNK).astype(jnp.int32)
    run = pl.kernel(
        _body,
        mesh=plsc.VectorSubcoreMesh(core_axis_name="c", subcore_axis_name="s"),
        compiler_params=pltpu.CompilerParams(use_tc_tiling_on_sc=False),
        out_type=jax.ShapeDtypeStruct((_N_ROWS, D_MODEL), jnp.float32),
        scratch_types=[
            pltpu.VMEM((_NCHUNK, _CHUNK), jnp.int32),
            pltpu.VMEM((SEQ, D_MODEL), jnp.float32),
            pltpu.VMEM((_NBUF, _CHUNK, D_MODEL), jnp.float32),
        ] + [pltpu.SemaphoreType.DMA] * (2 * _NBUF),
    )
    out = run(xi, table, pos)
    return out.reshape(BATCH, SEQ, D_MODEL)


def kernel(x, table):
    return _emb(x, table)


# speedup vs baseline: 1.6349x; 1.0994x over previous
import functools

import jax
import jax.numpy as jnp
from jax import lax
from jax.experimental import pallas as pl
from jax.experimental.pallas import tpu as pltpu
from jax.experimental.pallas import tpu_sc as plsc

D_MODEL = 128
BATCH = 1024
SEQ = 200

_NC = 2
_NS = 16
_NW = _NC * _NS
_N_ROWS = BATCH * SEQ
_ROWS_PER_W = _N_ROWS // _NW
_CHUNK = 100
_NCHUNK = _ROWS_PER_W // _CHUNK
_NBUF = 4
_L = 16


def _pos_encoding():
    pos = jnp.arange(SEQ, dtype=jnp.float32)[:, None]
    i_even = jnp.arange(0, D_MODEL, 2, dtype=jnp.float32)[None, :]
    angles = pos / jnp.power(10000.0, i_even / D_MODEL)
    enc = jnp.zeros((SEQ, D_MODEL), dtype=jnp.float32)
    enc = enc.at[:, 0::2].set(jnp.sin(angles))
    enc = enc.at[:, 1::2].set(jnp.cos(angles))
    return enc


def _body(idx_hbm, table_hbm, pos_hbm, out_hbm, idx_v, pos_v, rows_v, *sems):
    gsems = sems[:_NBUF]
    ssems = sems[_NBUF:]
    wid = lax.axis_index("s") * _NC + lax.axis_index("c")
    row0 = wid * _ROWS_PER_W

    pltpu.sync_copy(idx_hbm.at[pl.ds(wid * _NCHUNK, _NCHUNK)], idx_v)

    def start_gather(c, b):
        pltpu.make_async_copy(
            table_hbm.at[idx_v.at[c]], rows_v.at[b], gsems[b]
        ).start()

    def wait_gather(b):
        pltpu.make_async_copy(
            table_hbm.at[idx_v.at[0]], rows_v.at[b], gsems[b]
        ).wait()

    def start_store(c, b):
        pltpu.make_async_copy(
            rows_v.at[b], out_hbm.at[pl.ds(row0 + c * _CHUNK, _CHUNK)], ssems[b]
        ).start()

    def wait_store(b):
        pltpu.make_async_copy(
            rows_v.at[b], out_hbm.at[pl.ds(row0, _CHUNK)], ssems[b]
        ).wait()

    def add_pos(c, b):
        pbase = lax.rem(c, 2) * _CHUNK

        def add_row(r, carry2):
            for j in range(D_MODEL // _L):
                v = pos_v[pbase + r, pl.ds(j * _L, _L)]
                plsc.addupdate(rows_v.at[b, r, pl.ds(j * _L, _L)], v)
            return carry2

        lax.fori_loop(0, _CHUNK, add_row, 0, unroll=4)

    start_gather(0, 0)
    start_gather(1, 1)
    pltpu.sync_copy(pos_hbm, pos_v)

    for b in range(_NBUF):
        nxt = b + 2
        b2 = nxt % _NBUF
        if nxt < _NBUF:
            start_gather(nxt, b2)
        else:
            wait_store(b2)
            start_gather(nxt, b2)
        wait_gather(b)
        add_pos(b, b)
        start_store(b, b)

    def round_body(g, carry):
        for b in range(_NBUF):
            c = g * _NBUF + b
            nxt = c + 2
            b2 = (b + 2) % _NBUF

            @pl.when(nxt < _NCHUNK)
            def _():
                wait_store(b2)
                start_gather(nxt, b2)

            wait_gather(b)
            add_pos(c, b)
            start_store(c, b)

        return carry

    lax.fori_loop(1, _NCHUNK // _NBUF, round_body, 0)

    for b in range(_NBUF):
        wait_store(b)


@jax.jit
def _emb(x, table):
    pos = _pos_encoding()
    xi = x.reshape(_N_ROWS // _CHUNK, _CHUNK).astype(jnp.int32)
    run = pl.kernel(
        _body,
        mesh=plsc.VectorSubcoreMesh(core_axis_name="c", subcore_axis_name="s"),
        compiler_params=pltpu.CompilerParams(use_tc_tiling_on_sc=False),
        out_type=jax.ShapeDtypeStruct((_N_ROWS, D_MODEL), jnp.float32),
        scratch_types=[
            pltpu.VMEM((_NCHUNK, _CHUNK), jnp.int32),
            pltpu.VMEM((SEQ, D_MODEL), jnp.float32),
            pltpu.VMEM((_NBUF, _CHUNK, D_MODEL), jnp.float32),
        ] + [pltpu.SemaphoreType.DMA] * (2 * _NBUF),
    )
    out = run(xi, table, pos)
    return out.reshape(BATCH, SEQ, D_MODEL)


def kernel(x, table):
    return _emb(x, table)
